# baseline (speedup 1.0000x reference)
import functools

import jax
import jax.numpy as jnp
from jax import lax
from jax.experimental import pallas as pl
from jax.experimental.pallas import tpu as pltpu
from jax.experimental.pallas import tpu_sc as plsc

S = 2048
D = 768
H = 1024
E = 8
GS = 45
NG = 46
KSPAN = 46 * 48
INV = 2 * 8 * KSPAN
T_TILE = 256


def _router_body(x_ref, gk_ref, gb_ref, mi_ref, mf_ref, x16_ref):
    x = x_ref[...]
    logits = lax.dot_general(x, gk_ref[...], (((1,), (0,)), ((), ())),
                             preferred_element_type=jnp.float32)
    logits = logits + gb_ref[...]
    m = jnp.max(logits, axis=1, keepdims=True)
    ex = jnp.exp(logits - m)
    probs = ex / jnp.sum(ex, axis=1, keepdims=True)

    ioe = lax.broadcasted_iota(jnp.int32, (S, E), 1)
    m1 = jnp.max(probs, axis=1, keepdims=True)
    i1 = jnp.min(jnp.where(probs == m1, ioe, E), axis=1, keepdims=True)
    p2 = jnp.where(ioe == i1, -1.0, probs)
    m2 = jnp.max(p2, axis=1, keepdims=True)
    i2 = jnp.min(jnp.where(p2 == m2, ioe, E), axis=1, keepdims=True)

    io16 = lax.broadcasted_iota(jnp.int32, (S, 16), 1)
    M = ((io16 == i1) | (io16 == (i2 + 8))).astype(jnp.float32)

    ig_t = lax.broadcasted_iota(jnp.int32, (S, 48), 0)
    ig_g = lax.broadcasted_iota(jnp.int32, (S, 48), 1)
    G = ((ig_t >= GS * ig_g) & (ig_t < GS * ig_g + GS)).astype(jnp.float32)
    gidx = lax.broadcasted_iota(jnp.int32, (S, 48), 1).astype(jnp.float32)
    g_val = jnp.sum(G * gidx, axis=1, keepdims=True)
    g_start = g_val * GS

    tri_r = lax.broadcasted_iota(jnp.int32, (T_TILE, T_TILE), 0)
    tri_c = lax.broadcasted_iota(jnp.int32, (T_TILE, T_TILE), 1)
    parts = []
    for i in range(S // T_TILE):
        gs_i = g_start[i * T_TILE:(i + 1) * T_TILE, :]
        acc = jnp.zeros((T_TILE, 16), jnp.float32)
        for j in (i - 1, i):
            if j < 0:
                continue
            tp = (tri_c + j * T_TILE).astype(jnp.float32)
            t_abs = tri_r + i * T_TILE
            Lg = ((tri_c + j * T_TILE <= t_abs) & (tp >= gs_i)
                  ).astype(jnp.float32)
            Mj = M[j * T_TILE:(j + 1) * T_TILE, :]
            acc = acc + lax.dot_general(
                Lg, Mj, (((1,), (0,)), ((), ())),
                preferred_element_type=jnp.float32)
        parts.append(acc)
    posall = jnp.concatenate(parts, axis=0)
    pos1 = jnp.sum(M[:, :E] * posall[:, :E], axis=1, keepdims=True)
    pos2 = jnp.sum(M[:, E:] * posall[:, E:], axis=1, keepdims=True)

    e1f = i1.astype(jnp.float32)
    e2f = i2.astype(jnp.float32)
    s1 = e1f * KSPAN + g_val * 48 + pos1
    s2 = (8 * KSPAN) + e2f * KSPAN + g_val * 48 + pos2
    l1 = s1 + 8 * KSPAN
    l2 = s2 - 8 * KSPAN

    oh = lambda j: (lax.broadcasted_iota(jnp.int32, (S, E), 1) == j
                    ).astype(jnp.float32)
    mi = s1 * oh(0) + s2 * oh(1) + l1 * oh(2) + l2 * oh(3)
    mi_ref[...] = mi.astype(jnp.int32)
    mf_ref[...] = m1 * oh(0) + m2 * oh(1) + e1f * oh(2) + e2f * oh(3)
    x16_ref[...] = jnp.concatenate(
        [x.astype(jnp.bfloat16), jnp.zeros((8, D), jnp.bfloat16)], axis=0)


def _stage_a(x_flat, gate_kernel, gate_bias):
    return pl.pallas_call(
        _router_body,
        out_shape=[jax.ShapeDtypeStruct((S, E), jnp.int32),
                   jax.ShapeDtypeStruct((S, E), jnp.float32),
                   jax.ShapeDtypeStruct((S + 8, D), jnp.bfloat16)],
    )(x_flat, gate_kernel, gate_bias)


def _sc_body(s1_hbm, s2_hbm, l1_hbm, l2_hbm, xpad_hbm, xp1_hbm, xp2_hbm,
             zero_v, idx1_v, idx2_v, val_v, lidx1_v, lidx2_v, pval1_v,
             pval2_v, pidx1_v, pidx2_v, rows1_v, rows2_v, inv_sh,
             semA, semB, semC, semD):
    cid = lax.axis_index("c")
    sid = lax.axis_index("s")
    wid = sid * 2 + cid
    base = wid * 64

    c_s1 = pltpu.async_copy(s1_hbm.at[pl.ds(sid * 128, 128)], idx1_v, semA)
    c_s2 = pltpu.async_copy(s2_hbm.at[pl.ds(sid * 128, 128)], idx2_v, semB)
    c_l1 = pltpu.async_copy(l1_hbm.at[pl.ds(base, 64)], lidx1_v, semC)
    c_l2 = pltpu.async_copy(l2_hbm.at[pl.ds(base, 64)], lidx2_v, semD)

    zchunk = INV // 16
    for j in range(zchunk // 16):
        zero_v[pl.ds(j * 16, 16)] = jnp.zeros((16,), jnp.float32)
    for j in range(8):
        tno = sid * 128 + j * 16 + 1
        val_v[pl.ds(j * 16, 16)] = (
            lax.broadcasted_iota(jnp.int32, (16,), 0) + tno
        ).astype(jnp.float32)
    pltpu.sync_copy(zero_v, inv_sh.at[pl.ds(sid * zchunk, zchunk)])
    plsc.subcore_barrier()

    c_s1.wait()
    pltpu.sync_copy(val_v, inv_sh.at[idx1_v], add=True)
    c_s2.wait()
    pltpu.sync_copy(val_v, inv_sh.at[idx2_v], add=True)
    plsc.subcore_barrier()

    c_l1.wait()
    c_l2.wait()
    pltpu.sync_copy(inv_sh.at[lidx1_v], pval1_v)
    pltpu.sync_copy(inv_sh.at[lidx2_v], pval2_v)
    for pv, pi in ((pval1_v, pidx1_v), (pval2_v, pidx2_v)):
        for j in range(4):
            v = pv[pl.ds(j * 16, 16)]
            p = jnp.where(v == 0.0, jnp.float32(S), v - 1.0)
            pi[pl.ds(j * 16, 16)] = p.astype(jnp.int32)
    c_r1 = pltpu.async_copy(xpad_hbm.at[pidx1_v], rows1_v, semA)
    c_r2 = pltpu.async_copy(xpad_hbm.at[pidx2_v], rows2_v, semB)
    c_r1.wait()
    pltpu.sync_copy(rows1_v, xp1_hbm.at[pl.ds(base, 64)])
    c_r2.wait()
    pltpu.sync_copy(rows2_v, xp2_hbm.at[pl.ds(base, 64)])


@functools.lru_cache(maxsize=1)
def _make_sc_kernel():
    mesh = plsc.VectorSubcoreMesh(core_axis_name="c", subcore_axis_name="s")

    @functools.partial(
        pl.kernel,
        out_type=[jax.ShapeDtypeStruct((S, D // 2), jnp.float32),
                  jax.ShapeDtypeStruct((S, D // 2), jnp.float32)],
        mesh=mesh,
        scratch_types=[
            pltpu.VMEM((INV // 16,), jnp.float32),
            pltpu.VMEM((128,), jnp.int32),
            pltpu.VMEM((128,), jnp.int32),
            pltpu.VMEM((128,), jnp.float32),
            pltpu.VMEM((64,), jnp.int32),
            pltpu.VMEM((64,), jnp.int32),
            pltpu.VMEM((64,), jnp.float32),
            pltpu.VMEM((64,), jnp.float32),
            pltpu.VMEM((64,), jnp.int32),
            pltpu.VMEM((64,), jnp.int32),
            pltpu.VMEM((64, D // 2), jnp.float32),
            pltpu.VMEM((64, D // 2), jnp.float32),
            pltpu.VMEM_SHARED((INV,), jnp.float32),
            pltpu.SemaphoreType.DMA,
            pltpu.SemaphoreType.DMA,
            pltpu.SemaphoreType.DMA,
            pltpu.SemaphoreType.DMA,
        ],
    )
    def sc_partner_gather(s1, s2, l1, l2, xpad, xp1, xp2, *scratch):
        _sc_body(s1, s2, l1, l2, xpad, xp1, xp2, *scratch)

    return sc_partner_gather


def _sc_partner_gather(s1, s2, l1, l2, xpad):
    return _make_sc_kernel()(s1, s2, l1, l2, xpad)


def _ffn_body(x_ref, xp1_ref, xp2_ref, mf_ref, sk_ref, sv_ref, fk_ref,
              fv_ref, out_ref, k16_scr, v16_scr):
    e = pl.program_id(0)
    x = x_ref[0:S, :]

    @pl.when(e == 0)
    def _cast_shared():
        k16_scr[...] = sk_ref[0].astype(jnp.bfloat16)
        v16_scr[...] = sv_ref[0].astype(jnp.bfloat16)

    @pl.when(e > 0)
    def _cast_expert():
        k16_scr[...] = fk_ref[0].astype(jnp.bfloat16)
        v16_scr[...] = fv_ref[0].astype(jnp.bfloat16)

    @pl.when(e == 0)
    def _shared():
        h = lax.dot_general(x, k16_scr[...],
                            (((1,), (1,)), ((), ())),
                            preferred_element_type=jnp.float32)
        h = jax.nn.gelu(h)
        out_ref[...] = lax.dot_general(h.astype(jnp.bfloat16), v16_scr[...],
                                       (((1,), (1,)), ((), ())),
                                       preferred_element_type=jnp.float32)

    @pl.when(e > 0)
    def _expert():
        jf = (e - 1).astype(jnp.float32)
        m1 = mf_ref[:, 2:3] == jf
        m2 = mf_ref[:, 3:4] == jf
        zero16 = jnp.bfloat16(0)
        arg = (x + jnp.where(m1, xp1_ref[...], zero16)
               + jnp.where(m2, xp2_ref[...], zero16))
        h = lax.dot_general(arg, k16_scr[...],
                            (((1,), (1,)), ((), ())),
                            preferred_element_type=jnp.float32)
        h = jax.nn.gelu(h)
        y = lax.dot_general(h.astype(jnp.bfloat16), v16_scr[...],
                            (((1,), (1,)), ((), ())),
                            preferred_element_type=jnp.float32)
        w = (jnp.where(m1, mf_ref[:, 0:1], 0.0)
             + jnp.where(m2, mf_ref[:, 1:2], 0.0))
        out_ref[...] += y * w


def _stage_c(x_flat, xp1, xp2, mf, shared_keys, shared_values, ff_keys,
             ff_values):
    return pl.pallas_call(
        _ffn_body,
        grid=(E + 1,),
        in_specs=[
            pl.BlockSpec((S + 8, D), lambda e: (0, 0)),
            pl.BlockSpec((S, D), lambda e: (0, 0)),
            pl.BlockSpec((S, D), lambda e: (0, 0)),
            pl.BlockSpec((S, E), lambda e: (0, 0)),
            pl.BlockSpec((1, H, D), lambda e: (0, 0, 0)),
            pl.BlockSpec((1, D, H), lambda e: (0, 0, 0)),
            pl.BlockSpec((1, H, D), lambda e: (jnp.maximum(e - 1, 0), 0, 0)),
            pl.BlockSpec((1, D, H), lambda e: (jnp.maximum(e - 1, 0), 0, 0)),
        ],
        out_specs=pl.BlockSpec((S, D), lambda e: (0, 0)),
        out_shape=jax.ShapeDtypeStruct((S, D), jnp.float32),
        scratch_shapes=[pltpu.VMEM((H, D), jnp.bfloat16),
                        pltpu.VMEM((D, H), jnp.bfloat16)],
        name="moe_ffn",
        compiler_params=pltpu.CompilerParams(
            dimension_semantics=("arbitrary",)),
    )(x_flat, xp1, xp2, mf, shared_keys, shared_values, ff_keys, ff_values)


def kernel(x, gate_kernel, gate_bias, ff_keys, ff_values, shared_keys,
           shared_values):
    B, Sx, Dx = x.shape
    x_flat = x.reshape(S, D)

    mi, mf, xpad16 = _stage_a(x_flat, gate_kernel, gate_bias)
    s1 = mi[:, 0]
    s2 = mi[:, 1]
    l1 = mi[:, 2]
    l2 = mi[:, 3]

    xpad32 = lax.bitcast_convert_type(
        xpad16.reshape(S + 8, D // 2, 2), jnp.float32)
    xp1_32, xp2_32 = _sc_partner_gather(s1, s2, l1, l2, xpad32)
    xp1 = lax.bitcast_convert_type(xp1_32, jnp.bfloat16).reshape(S, D)
    xp2 = lax.bitcast_convert_type(xp2_32, jnp.bfloat16).reshape(S, D)

    out = _stage_c(xpad16, xp1, xp2, mf, shared_keys, shared_values,
                   ff_keys, ff_values)
    return out.reshape(B, Sx, Dx)


# speedup vs baseline: 1.4225x; 1.4225x over previous
import functools

import jax
import jax.numpy as jnp
from jax import lax
from jax.experimental import pallas as pl
from jax.experimental.pallas import tpu as pltpu
from jax.experimental.pallas import tpu_sc as plsc

S = 2048
D = 768
H = 1024
E = 8
GS = 45
NG = 46
KSPAN = 46 * 48
INV = 2 * 8 * KSPAN
T_TILE = 256


def _router_body(x_ref, gk_ref, gb_ref, mi_ref, mf_ref, x16_ref):
    x = x_ref[...]
    logits = lax.dot_general(x, gk_ref[...], (((1,), (0,)), ((), ())),
                             preferred_element_type=jnp.float32)
    logits = logits + gb_ref[...]
    m = jnp.max(logits, axis=1, keepdims=True)
    ex = jnp.exp(logits - m)
    probs = ex / jnp.sum(ex, axis=1, keepdims=True)

    ioe = lax.broadcasted_iota(jnp.int32, (S, E), 1)
    m1 = jnp.max(probs, axis=1, keepdims=True)
    i1 = jnp.min(jnp.where(probs == m1, ioe, E), axis=1, keepdims=True)
    p2 = jnp.where(ioe == i1, -1.0, probs)
    m2 = jnp.max(p2, axis=1, keepdims=True)
    i2 = jnp.min(jnp.where(p2 == m2, ioe, E), axis=1, keepdims=True)

    io16 = lax.broadcasted_iota(jnp.int32, (S, 16), 1)
    M = ((io16 == i1) | (io16 == (i2 + 8))).astype(jnp.float32)

    ig_t = lax.broadcasted_iota(jnp.int32, (S, 48), 0)
    ig_g = lax.broadcasted_iota(jnp.int32, (S, 48), 1)
    G = ((ig_t >= GS * ig_g) & (ig_t < GS * ig_g + GS)).astype(jnp.float32)
    gidx = lax.broadcasted_iota(jnp.int32, (S, 48), 1).astype(jnp.float32)
    g_val = jnp.sum(G * gidx, axis=1, keepdims=True)
    g_start = g_val * GS

    tri_r = lax.broadcasted_iota(jnp.int32, (T_TILE, T_TILE), 0)
    tri_c = lax.broadcasted_iota(jnp.int32, (T_TILE, T_TILE), 1)
    parts = []
    for i in range(S // T_TILE):
        gs_i = g_start[i * T_TILE:(i + 1) * T_TILE, :]
        acc = jnp.zeros((T_TILE, 16), jnp.float32)
        for j in (i - 1, i):
            if j < 0:
                continue
            tp = (tri_c + j * T_TILE).astype(jnp.float32)
            t_abs = tri_r + i * T_TILE
            Lg = ((tri_c + j * T_TILE <= t_abs) & (tp >= gs_i)
                  ).astype(jnp.float32)
            Mj = M[j * T_TILE:(j + 1) * T_TILE, :]
            acc = acc + lax.dot_general(
                Lg, Mj, (((1,), (0,)), ((), ())),
                preferred_element_type=jnp.float32)
        parts.append(acc)
    posall = jnp.concatenate(parts, axis=0)
    pos1 = jnp.sum(M[:, :E] * posall[:, :E], axis=1, keepdims=True)
    pos2 = jnp.sum(M[:, E:] * posall[:, E:], axis=1, keepdims=True)

    e1f = i1.astype(jnp.float32)
    e2f = i2.astype(jnp.float32)
    s1 = e1f * KSPAN + g_val * 48 + pos1
    s2 = (8 * KSPAN) + e2f * KSPAN + g_val * 48 + pos2
    l1 = s1 + 8 * KSPAN
    l2 = s2 - 8 * KSPAN

    oh = lambda j: (lax.broadcasted_iota(jnp.int32, (S, E), 1) == j
                    ).astype(jnp.float32)
    mi = s1 * oh(0) + s2 * oh(1) + l1 * oh(2) + l2 * oh(3)
    mi_ref[...] = mi.astype(jnp.int32)
    mf_ref[...] = m1 * oh(0) + m2 * oh(1) + e1f * oh(2) + e2f * oh(3)
    x16_ref[...] = jnp.concatenate(
        [x.astype(jnp.bfloat16), jnp.zeros((8, D), jnp.bfloat16)], axis=0)


def _stage_a(x_flat, gate_kernel, gate_bias):
    return pl.pallas_call(
        _router_body,
        out_shape=[jax.ShapeDtypeStruct((S, E), jnp.int32),
                   jax.ShapeDtypeStruct((S, E), jnp.float32),
                   jax.ShapeDtypeStruct((S + 8, D), jnp.bfloat16)],
    )(x_flat, gate_kernel, gate_bias)


def _sc_body(s1_hbm, s2_hbm, l1_hbm, l2_hbm, xpad_hbm, xp1_hbm, xp2_hbm,
             zero_v, idx1_v, idx2_v, val_v, lidx1_v, lidx2_v, pval1_v,
             pval2_v, pidx1_v, pidx2_v, rows1_v, rows2_v, inv_sh,
             semA, semB, semC, semD):
    cid = lax.axis_index("c")
    sid = lax.axis_index("s")
    wid = sid * 2 + cid
    base = wid * 64

    c_s1 = pltpu.async_copy(s1_hbm.at[pl.ds(sid * 128, 128)], idx1_v, semA)
    c_s2 = pltpu.async_copy(s2_hbm.at[pl.ds(sid * 128, 128)], idx2_v, semB)
    c_l1 = pltpu.async_copy(l1_hbm.at[pl.ds(base, 64)], lidx1_v, semC)
    c_l2 = pltpu.async_copy(l2_hbm.at[pl.ds(base, 64)], lidx2_v, semD)

    zchunk = INV // 16
    for j in range(zchunk // 16):
        zero_v[pl.ds(j * 16, 16)] = jnp.zeros((16,), jnp.float32)
    for j in range(8):
        tno = sid * 128 + j * 16 + 1
        val_v[pl.ds(j * 16, 16)] = (
            lax.broadcasted_iota(jnp.int32, (16,), 0) + tno
        ).astype(jnp.float32)
    pltpu.sync_copy(zero_v, inv_sh.at[pl.ds(sid * zchunk, zchunk)])
    plsc.subcore_barrier()

    c_s1.wait()
    pltpu.sync_copy(val_v, inv_sh.at[idx1_v], add=True)
    c_s2.wait()
    pltpu.sync_copy(val_v, inv_sh.at[idx2_v], add=True)
    plsc.subcore_barrier()

    c_l1.wait()
    c_l2.wait()
    pltpu.sync_copy(inv_sh.at[lidx1_v], pval1_v)
    pltpu.sync_copy(inv_sh.at[lidx2_v], pval2_v)
    for pv, pi in ((pval1_v, pidx1_v), (pval2_v, pidx2_v)):
        for j in range(4):
            v = pv[pl.ds(j * 16, 16)]
            p = jnp.where(v == 0.0, jnp.float32(S), v - 1.0)
            pi[pl.ds(j * 16, 16)] = p.astype(jnp.int32)
    c_r1 = pltpu.async_copy(xpad_hbm.at[pidx1_v], rows1_v, semA)
    c_r2 = pltpu.async_copy(xpad_hbm.at[pidx2_v], rows2_v, semB)
    c_r1.wait()
    pltpu.sync_copy(rows1_v, xp1_hbm.at[pl.ds(base, 64)])
    c_r2.wait()
    pltpu.sync_copy(rows2_v, xp2_hbm.at[pl.ds(base, 64)])


@functools.lru_cache(maxsize=1)
def _make_sc_kernel():
    mesh = plsc.VectorSubcoreMesh(core_axis_name="c", subcore_axis_name="s")

    @functools.partial(
        pl.kernel,
        out_type=[jax.ShapeDtypeStruct((S, D), jnp.float32),
                  jax.ShapeDtypeStruct((S, D), jnp.float32)],
        mesh=mesh,
        scratch_types=[
            pltpu.VMEM((INV // 16,), jnp.float32),
            pltpu.VMEM((128,), jnp.int32),
            pltpu.VMEM((128,), jnp.int32),
            pltpu.VMEM((128,), jnp.float32),
            pltpu.VMEM((64,), jnp.int32),
            pltpu.VMEM((64,), jnp.int32),
            pltpu.VMEM((64,), jnp.float32),
            pltpu.VMEM((64,), jnp.float32),
            pltpu.VMEM((64,), jnp.int32),
            pltpu.VMEM((64,), jnp.int32),
            pltpu.VMEM((64, D), jnp.float32),
            pltpu.VMEM((64, D), jnp.float32),
            pltpu.VMEM_SHARED((INV,), jnp.float32),
            pltpu.SemaphoreType.DMA,
            pltpu.SemaphoreType.DMA,
            pltpu.SemaphoreType.DMA,
            pltpu.SemaphoreType.DMA,
        ],
    )
    def sc_partner_gather(s1, s2, l1, l2, xpad, xp1, xp2, *scratch):
        _sc_body(s1, s2, l1, l2, xpad, xp1, xp2, *scratch)

    return sc_partner_gather


def _sc_partner_gather(s1, s2, l1, l2, xpad):
    return _make_sc_kernel()(s1, s2, l1, l2, xpad)


def _ffn_body(x_ref, xp1_ref, xp2_ref, mf_ref, sk_ref, sv_ref, fk_ref,
              fv_ref, out_ref, k16_scr, v16_scr):
    e = pl.program_id(0)
    x = x_ref[0:S, :]

    @pl.when(e == 0)
    def _cast_shared():
        k16_scr[...] = sk_ref[0].astype(jnp.bfloat16)
        v16_scr[...] = sv_ref[0].astype(jnp.bfloat16)

    @pl.when(e > 0)
    def _cast_expert():
        k16_scr[...] = fk_ref[0].astype(jnp.bfloat16)
        v16_scr[...] = fv_ref[0].astype(jnp.bfloat16)

    @pl.when(e == 0)
    def _shared():
        h = lax.dot_general(x, k16_scr[...],
                            (((1,), (1,)), ((), ())),
                            preferred_element_type=jnp.float32)
        h = jax.nn.gelu(h)
        out_ref[...] = lax.dot_general(h.astype(jnp.bfloat16), v16_scr[...],
                                       (((1,), (1,)), ((), ())),
                                       preferred_element_type=jnp.float32)

    @pl.when(e > 0)
    def _expert():
        jf = (e - 1).astype(jnp.float32)
        m1 = mf_ref[:, 2:3] == jf
        m2 = mf_ref[:, 3:4] == jf
        zero16 = jnp.bfloat16(0)
        arg = (x + jnp.where(m1, xp1_ref[...], zero16)
               + jnp.where(m2, xp2_ref[...], zero16))
        h = lax.dot_general(arg, k16_scr[...],
                            (((1,), (1,)), ((), ())),
                            preferred_element_type=jnp.float32)
        h = jax.nn.gelu(h)
        y = lax.dot_general(h.astype(jnp.bfloat16), v16_scr[...],
                            (((1,), (1,)), ((), ())),
                            preferred_element_type=jnp.float32)
        w = (jnp.where(m1, mf_ref[:, 0:1], 0.0)
             + jnp.where(m2, mf_ref[:, 1:2], 0.0))
        out_ref[...] += y * w


def _stage_c(x_flat, xp1, xp2, mf, shared_keys, shared_values, ff_keys,
             ff_values):
    return pl.pallas_call(
        _ffn_body,
        grid=(E + 1,),
        in_specs=[
            pl.BlockSpec((S + 8, D), lambda e: (0, 0)),
            pl.BlockSpec((S, D), lambda e: (0, 0)),
            pl.BlockSpec((S, D), lambda e: (0, 0)),
            pl.BlockSpec((S, E), lambda e: (0, 0)),
            pl.BlockSpec((1, H, D), lambda e: (0, 0, 0)),
            pl.BlockSpec((1, D, H), lambda e: (0, 0, 0)),
            pl.BlockSpec((1, H, D), lambda e: (jnp.maximum(e - 1, 0), 0, 0)),
            pl.BlockSpec((1, D, H), lambda e: (jnp.maximum(e - 1, 0), 0, 0)),
        ],
        out_specs=pl.BlockSpec((S, D), lambda e: (0, 0)),
        out_shape=jax.ShapeDtypeStruct((S, D), jnp.float32),
        scratch_shapes=[pltpu.VMEM((H, D), jnp.bfloat16),
                        pltpu.VMEM((D, H), jnp.bfloat16)],
        name="moe_ffn",
        compiler_params=pltpu.CompilerParams(
            dimension_semantics=("arbitrary",)),
    )(x_flat, xp1, xp2, mf, shared_keys, shared_values, ff_keys, ff_values)


def kernel(x, gate_kernel, gate_bias, ff_keys, ff_values, shared_keys,
           shared_values):
    B, Sx, Dx = x.shape
    x_flat = x.reshape(S, D)

    mi, mf, xpad16 = _stage_a(x_flat, gate_kernel, gate_bias)
    s1 = mi[:, 0]
    s2 = mi[:, 1]
    l1 = mi[:, 2]
    l2 = mi[:, 3]

    x_pad = jnp.concatenate(
        [x_flat, jnp.zeros((8, D), jnp.float32)], axis=0)
    xp1_32, xp2_32 = _sc_partner_gather(s1, s2, l1, l2, x_pad)
    xp1 = xp1_32.astype(jnp.bfloat16)
    xp2 = xp2_32.astype(jnp.bfloat16)

    out = _stage_c(xpad16, xp1, xp2, mf, shared_keys, shared_values,
                   ff_keys, ff_values)
    return out.reshape(B, Sx, Dx)


# speedup vs baseline: 1.4700x; 1.0334x over previous
import functools

import jax
import jax.numpy as jnp
from jax import lax
from jax.experimental import pallas as pl
from jax.experimental.pallas import tpu as pltpu
from jax.experimental.pallas import tpu_sc as plsc

S = 2048
D = 768
H = 1024
E = 8
GS = 45
NG = 46
KSPAN = 46 * 48
INV = 2 * 8 * KSPAN
T_TILE = 256


def _router_body(x_ref, gk_ref, gb_ref, mi_ref, mf_ref, x16_ref):
    x = x_ref[...]
    logits = lax.dot_general(x, gk_ref[...], (((1,), (0,)), ((), ())),
                             preferred_element_type=jnp.float32)
    logits = logits + gb_ref[...]
    m = jnp.max(logits, axis=1, keepdims=True)
    ex = jnp.exp(logits - m)
    probs = ex / jnp.sum(ex, axis=1, keepdims=True)

    ioe = lax.broadcasted_iota(jnp.int32, (S, E), 1)
    m1 = jnp.max(probs, axis=1, keepdims=True)
    i1 = jnp.min(jnp.where(probs == m1, ioe, E), axis=1, keepdims=True)
    p2 = jnp.where(ioe == i1, -1.0, probs)
    m2 = jnp.max(p2, axis=1, keepdims=True)
    i2 = jnp.min(jnp.where(p2 == m2, ioe, E), axis=1, keepdims=True)

    io16 = lax.broadcasted_iota(jnp.int32, (S, 16), 1)
    M = ((io16 == i1) | (io16 == (i2 + 8))).astype(jnp.float32)

    ig_t = lax.broadcasted_iota(jnp.int32, (S, 48), 0)
    ig_g = lax.broadcasted_iota(jnp.int32, (S, 48), 1)
    G = ((ig_t >= GS * ig_g) & (ig_t < GS * ig_g + GS)).astype(jnp.float32)
    gidx = lax.broadcasted_iota(jnp.int32, (S, 48), 1).astype(jnp.float32)
    g_val = jnp.sum(G * gidx, axis=1, keepdims=True)
    g_start = g_val * GS

    tri_r = lax.broadcasted_iota(jnp.int32, (T_TILE, T_TILE), 0)
    tri_c = lax.broadcasted_iota(jnp.int32, (T_TILE, T_TILE), 1)
    parts = []
    for i in range(S // T_TILE):
        gs_i = g_start[i * T_TILE:(i + 1) * T_TILE, :]
        acc = jnp.zeros((T_TILE, 16), jnp.float32)
        for j in (i - 1, i):
            if j < 0:
                continue
            tp = (tri_c + j * T_TILE).astype(jnp.float32)
            t_abs = tri_r + i * T_TILE
            Lg = ((tri_c + j * T_TILE <= t_abs) & (tp >= gs_i)
                  ).astype(jnp.float32)
            Mj = M[j * T_TILE:(j + 1) * T_TILE, :]
            acc = acc + lax.dot_general(
                Lg, Mj, (((1,), (0,)), ((), ())),
                preferred_element_type=jnp.float32)
        parts.append(acc)
    posall = jnp.concatenate(parts, axis=0)
    pos1 = jnp.sum(M[:, :E] * posall[:, :E], axis=1, keepdims=True)
    pos2 = jnp.sum(M[:, E:] * posall[:, E:], axis=1, keepdims=True)

    e1f = i1.astype(jnp.float32)
    e2f = i2.astype(jnp.float32)
    s1 = e1f * KSPAN + g_val * 48 + pos1
    s2 = (8 * KSPAN) + e2f * KSPAN + g_val * 48 + pos2
    l1 = s1 + 8 * KSPAN
    l2 = s2 - 8 * KSPAN

    oh = lambda j: (lax.broadcasted_iota(jnp.int32, (S, E), 1) == j
                    ).astype(jnp.float32)
    mi = s1 * oh(0) + s2 * oh(1) + l1 * oh(2) + l2 * oh(3)
    mi_ref[...] = mi.astype(jnp.int32)
    mf_ref[...] = m1 * oh(0) + m2 * oh(1) + e1f * oh(2) + e2f * oh(3)
    x16_ref[...] = jnp.concatenate(
        [x.astype(jnp.bfloat16), jnp.zeros((8, D), jnp.bfloat16)], axis=0)


def _stage_a(x_flat, gate_kernel, gate_bias):
    return pl.pallas_call(
        _router_body,
        out_shape=[jax.ShapeDtypeStruct((S, E), jnp.int32),
                   jax.ShapeDtypeStruct((S, E), jnp.float32),
                   jax.ShapeDtypeStruct((S + 8, D), jnp.bfloat16)],
    )(x_flat, gate_kernel, gate_bias)


def _sc_body(s1_hbm, s2_hbm, l1_hbm, l2_hbm, xpad_hbm, xp1_hbm, xp2_hbm,
             zero_v, idx1_v, idx2_v, val_v, lidx1_v, lidx2_v, pval1_v,
             pval2_v, pidx1_v, pidx2_v, rows1_v, rows2_v, inv_sh,
             semA, semB, semC, semD):
    cid = lax.axis_index("c")
    sid = lax.axis_index("s")
    wid = sid * 2 + cid
    base = wid * 64

    c_s1 = pltpu.async_copy(s1_hbm.at[pl.ds(sid * 128, 128)], idx1_v, semA)
    c_s2 = pltpu.async_copy(s2_hbm.at[pl.ds(sid * 128, 128)], idx2_v, semB)
    c_l1 = pltpu.async_copy(l1_hbm.at[pl.ds(base, 64)], lidx1_v, semC)
    c_l2 = pltpu.async_copy(l2_hbm.at[pl.ds(base, 64)], lidx2_v, semD)

    zchunk = INV // 16
    for j in range(zchunk // 16):
        zero_v[pl.ds(j * 16, 16)] = jnp.zeros((16,), jnp.float32)
    for j in range(8):
        tno = sid * 128 + j * 16 + 1
        val_v[pl.ds(j * 16, 16)] = (
            lax.broadcasted_iota(jnp.int32, (16,), 0) + tno
        ).astype(jnp.float32)
    pltpu.sync_copy(zero_v, inv_sh.at[pl.ds(sid * zchunk, zchunk)])
    plsc.subcore_barrier()

    c_s1.wait()
    pltpu.sync_copy(val_v, inv_sh.at[idx1_v], add=True)
    c_s2.wait()
    pltpu.sync_copy(val_v, inv_sh.at[idx2_v], add=True)
    plsc.subcore_barrier()

    c_l1.wait()
    c_l2.wait()
    pltpu.sync_copy(inv_sh.at[lidx1_v], pval1_v)
    pltpu.sync_copy(inv_sh.at[lidx2_v], pval2_v)
    for pv, pi in ((pval1_v, pidx1_v), (pval2_v, pidx2_v)):
        for j in range(4):
            v = pv[pl.ds(j * 16, 16)]
            p = jnp.where(v == 0.0, jnp.float32(S), v - 1.0)
            pi[pl.ds(j * 16, 16)] = p.astype(jnp.int32)
    c_r1 = pltpu.async_copy(xpad_hbm.at[pidx1_v], rows1_v, semA)
    c_r2 = pltpu.async_copy(xpad_hbm.at[pidx2_v], rows2_v, semB)
    c_r1.wait()
    pltpu.sync_copy(rows1_v, xp1_hbm.at[pl.ds(base, 64)])
    c_r2.wait()
    pltpu.sync_copy(rows2_v, xp2_hbm.at[pl.ds(base, 64)])


@functools.lru_cache(maxsize=1)
def _make_sc_kernel():
    mesh = plsc.VectorSubcoreMesh(core_axis_name="c", subcore_axis_name="s")

    @functools.partial(
        pl.kernel,
        out_type=[jax.ShapeDtypeStruct((S, D), jnp.float32),
                  jax.ShapeDtypeStruct((S, D), jnp.float32)],
        mesh=mesh,
        scratch_types=[
            pltpu.VMEM((INV // 16,), jnp.float32),
            pltpu.VMEM((128,), jnp.int32),
            pltpu.VMEM((128,), jnp.int32),
            pltpu.VMEM((128,), jnp.float32),
            pltpu.VMEM((64,), jnp.int32),
            pltpu.VMEM((64,), jnp.int32),
            pltpu.VMEM((64,), jnp.float32),
            pltpu.VMEM((64,), jnp.float32),
            pltpu.VMEM((64,), jnp.int32),
            pltpu.VMEM((64,), jnp.int32),
            pltpu.VMEM((64, D), jnp.float32),
            pltpu.VMEM((64, D), jnp.float32),
            pltpu.VMEM_SHARED((INV,), jnp.float32),
            pltpu.SemaphoreType.DMA,
            pltpu.SemaphoreType.DMA,
            pltpu.SemaphoreType.DMA,
            pltpu.SemaphoreType.DMA,
        ],
    )
    def sc_partner_gather(s1, s2, l1, l2, xpad, xp1, xp2, *scratch):
        _sc_body(s1, s2, l1, l2, xpad, xp1, xp2, *scratch)

    return sc_partner_gather


def _sc_partner_gather(s1, s2, l1, l2, xpad):
    return _make_sc_kernel()(s1, s2, l1, l2, xpad)


def _shared_body(x_ref, sk_ref, sv_ref, sh_ref, k16_scr, v16_scr):
    k16_scr[...] = sk_ref[0].astype(jnp.bfloat16)
    v16_scr[...] = sv_ref[0].astype(jnp.bfloat16)
    h = lax.dot_general(x_ref[0:S, :], k16_scr[...],
                        (((1,), (1,)), ((), ())),
                        preferred_element_type=jnp.float32)
    h = jax.nn.gelu(h)
    sh_ref[...] = lax.dot_general(
        h.astype(jnp.bfloat16), v16_scr[...], (((1,), (1,)), ((), ())),
        preferred_element_type=jnp.float32).astype(jnp.bfloat16)


def _stage_shared(xpad16, shared_keys, shared_values):
    return pl.pallas_call(
        _shared_body,
        out_shape=jax.ShapeDtypeStruct((S, D), jnp.bfloat16),
        scratch_shapes=[pltpu.VMEM((H, D), jnp.bfloat16),
                        pltpu.VMEM((D, H), jnp.bfloat16)],
    )(xpad16, shared_keys, shared_values)


def _ffn_body(x_ref, xp1_ref, xp2_ref, mf_ref, sh_ref, fk_ref,
              fv_ref, out_ref, k16_scr, v16_scr):
    e = pl.program_id(0)
    x = x_ref[0:S, :]

    k16_scr[...] = fk_ref[0].astype(jnp.bfloat16)
    v16_scr[...] = fv_ref[0].astype(jnp.bfloat16)

    jf = e.astype(jnp.float32)
    m1 = mf_ref[:, 2:3] == jf
    m2 = mf_ref[:, 3:4] == jf
    zero16 = jnp.bfloat16(0)
    arg = (x + jnp.where(m1, xp1_ref[...], zero16)
           + jnp.where(m2, xp2_ref[...], zero16))
    h = lax.dot_general(arg, k16_scr[...],
                        (((1,), (1,)), ((), ())),
                        preferred_element_type=jnp.float32)
    h = jax.nn.gelu(h)
    y = lax.dot_general(h.astype(jnp.bfloat16), v16_scr[...],
                        (((1,), (1,)), ((), ())),
                        preferred_element_type=jnp.float32)
    w = (jnp.where(m1, mf_ref[:, 0:1], 0.0)
         + jnp.where(m2, mf_ref[:, 1:2], 0.0))

    @pl.when(e == 0)
    def _init():
        out_ref[...] = sh_ref[...].astype(jnp.float32) + y * w

    @pl.when(e > 0)
    def _acc():
        out_ref[...] += y * w


def _stage_c(xpad16, xp1, xp2, mf, sh16, ff_keys, ff_values):
    return pl.pallas_call(
        _ffn_body,
        grid=(E,),
        in_specs=[
            pl.BlockSpec((S + 8, D), lambda e: (0, 0)),
            pl.BlockSpec((S, D), lambda e: (0, 0)),
            pl.BlockSpec((S, D), lambda e: (0, 0)),
            pl.BlockSpec((S, E), lambda e: (0, 0)),
            pl.BlockSpec((S, D), lambda e: (0, 0)),
            pl.BlockSpec((1, H, D), lambda e: (e, 0, 0)),
            pl.BlockSpec((1, D, H), lambda e: (e, 0, 0)),
        ],
        out_specs=pl.BlockSpec((S, D), lambda e: (0, 0)),
        out_shape=jax.ShapeDtypeStruct((S, D), jnp.float32),
        scratch_shapes=[pltpu.VMEM((H, D), jnp.bfloat16),
                        pltpu.VMEM((D, H), jnp.bfloat16)],
        name="moe_ffn",
        compiler_params=pltpu.CompilerParams(
            dimension_semantics=("arbitrary",)),
    )(xpad16, xp1, xp2, mf, sh16, ff_keys, ff_values)


def kernel(x, gate_kernel, gate_bias, ff_keys, ff_values, shared_keys,
           shared_values):
    B, Sx, Dx = x.shape
    x_flat = x.reshape(S, D)

    mi, mf, xpad16 = _stage_a(x_flat, gate_kernel, gate_bias)
    s1 = mi[:, 0]
    s2 = mi[:, 1]
    l1 = mi[:, 2]
    l2 = mi[:, 3]

    x_pad = jnp.concatenate(
        [x_flat, jnp.zeros((8, D), jnp.float32)], axis=0)
    xp1_32, xp2_32 = _sc_partner_gather(s1, s2, l1, l2, x_pad)
    sh16 = _stage_shared(xpad16, shared_keys, shared_values)
    xp1 = xp1_32.astype(jnp.bfloat16)
    xp2 = xp2_32.astype(jnp.bfloat16)

    out = _stage_c(xpad16, xp1, xp2, mf, sh16, ff_keys, ff_values)
    return out.reshape(B, Sx, Dx)
